# baseline (speedup 1.0000x reference)
import functools

import jax
import jax.numpy as jnp
from jax import lax
from jax.experimental import pallas as pl
from jax.experimental.pallas import tpu as pltpu
from jax.experimental.pallas import tpu_sc as plsc

B, T, Y, X = 8, 12, 448, 304
NW = 32
IMGS_PER_W = (B * T) // NW
YW = 64
NCHUNK = Y // YW
NVEC = X // 16
THR = 0.15

_mesh = plsc.VectorSubcoreMesh(core_axis_name="c", subcore_axis_name="s")


@functools.partial(
    pl.kernel,
    out_type=jax.ShapeDtypeStruct((NW, 4, 16), jnp.float32),
    mesh=_mesh,
    scratch_types=[
        pltpu.VMEM((YW, X), jnp.float32),
        pltpu.VMEM((YW, X), jnp.float32),
        pltpu.VMEM((YW, X), jnp.float32),
        pltpu.VMEM((4, 16), jnp.float32),
    ],
)
def _sc_masked_sums(p_hbm, t_hbm, m_hbm, out_hbm, pbuf, tbuf, mbuf, obuf):
    wid = lax.axis_index("s") * 2 + lax.axis_index("c")

    zf = jnp.zeros((16,), jnp.float32)
    thr = jnp.float32(THR)

    def row_body(y, carry):
        acc = list(carry)
        for j in range(NVEC):
            pj = pbuf[y, pl.ds(16 * j, 16)]
            tj = tbuf[y, pl.ds(16 * j, 16)]
            mj = mbuf[y, pl.ds(16 * j, 16)]
            d = pj - tj
            dm = d * mj
            x = (pj >= thr) != (tj >= thr)
            s = j % 2
            aabs, asq, cnt, xor = acc[4 * s:4 * s + 4]
            acc[4 * s] = aabs + jnp.abs(dm)
            acc[4 * s + 1] = asq + dm * dm
            acc[4 * s + 2] = cnt + mj
            acc[4 * s + 3] = xor + jnp.where(x, mj, jnp.float32(0.0))
        return tuple(acc)

    def chunk_body(k, tot):
        img = wid * IMGS_PER_W + k // NCHUNK
        b = (img * 43691) >> 19
        tt = img - b * T
        y0 = pl.multiple_of((k % NCHUNK) * YW, YW)
        accs = lax.fori_loop(0, YW, row_body, (zf,) * 8)
        return tuple(t + a for t, a in zip(tot, accs))

    tots = lax.fori_loop(
        0, IMGS_PER_W * NCHUNK, chunk_body, (zf,) * 8)
    t_abs, t_sq, t_cnt, t_xor = (tots[i] + tots[4 + i] for i in range(4))

    obuf[0, :] = t_cnt
    obuf[1, :] = t_abs
    obuf[2, :] = t_sq
    obuf[3, :] = t_xor
    pltpu.sync_copy(obuf, out_hbm.at[wid])


def kernel(preds, target, mask):
    mf = mask.astype(jnp.float32)

    parts = _sc_masked_sums(preds, target, mf)

    cnt = jnp.sum(parts[:, 0, :])
    sabs = jnp.sum(parts[:, 1, :])
    ssq = jnp.sum(parts[:, 2, :])
    xcnt = jnp.sum(parts[:, 3, :])

    masked_mae = sabs / cnt
    masked_mse = ssq / cnt
    masked_rmse = jnp.sqrt(masked_mse)
    iiee = xcnt * jnp.float32(625.0 / 1000000.0) / jnp.float32(96.0)
    bacc = jnp.float32(1.0) - iiee / jnp.float32(27207.0 * 625.0 / 1000000.0)
    return (masked_mae, masked_rmse, iiee, bacc, masked_mae)


# speedup vs baseline: 1.0005x; 1.0005x over previous
import functools

import jax
import jax.numpy as jnp
from jax import lax
from jax.experimental import pallas as pl
from jax.experimental.pallas import tpu as pltpu
from jax.experimental.pallas import tpu_sc as plsc

B, T, Y, X = 8, 12, 448, 304
NW = 32
IMGS_PER_W = (B * T) // NW
YW = 32
NCHUNK = Y // YW
NVEC = X // 16
THR = 0.15

_mesh = plsc.VectorSubcoreMesh(core_axis_name="c", subcore_axis_name="s")


@functools.partial(
    pl.kernel,
    out_type=jax.ShapeDtypeStruct((NW, 4, 16), jnp.float32),
    mesh=_mesh,
    scratch_types=[
        pltpu.VMEM((YW, X), jnp.float32),
        pltpu.VMEM((YW, X), jnp.float32),
        pltpu.VMEM((YW, X), jnp.float32),
        pltpu.VMEM((YW, X), jnp.float32),
        pltpu.VMEM((YW, X), jnp.float32),
        pltpu.VMEM((YW, X), jnp.float32),
        pltpu.VMEM((4, 16), jnp.float32),
        pltpu.SemaphoreType.DMA,
        pltpu.SemaphoreType.DMA,
    ],
)
def _sc_masked_sums(p_hbm, t_hbm, m_hbm, out_hbm,
                    pbuf0, pbuf1, tbuf0, tbuf1, mbuf0, mbuf1, obuf,
                    sem0, sem1):
    bufs = ((pbuf0, tbuf0, mbuf0), (pbuf1, tbuf1, mbuf1))
    sems = (sem0, sem1)
    wid = lax.axis_index("s") * 2 + lax.axis_index("c")

    zf = jnp.zeros((16,), jnp.float32)
    thr = jnp.float32(THR)

    def row_body_for(pbuf, tbuf, mbuf):
      def row_body(y, carry):
        acc = list(carry)
        for j in range(NVEC):
            pj = pbuf[y, pl.ds(16 * j, 16)]
            tj = tbuf[y, pl.ds(16 * j, 16)]
            mj = mbuf[y, pl.ds(16 * j, 16)]
            d = pj - tj
            dm = d * mj
            x = (pj >= thr) != (tj >= thr)
            s = j % 2
            aabs, asq, cnt, xor = acc[4 * s:4 * s + 4]
            acc[4 * s] = aabs + jnp.abs(dm)
            acc[4 * s + 1] = asq + dm * dm
            acc[4 * s + 2] = cnt + mj
            acc[4 * s + 3] = xor + jnp.where(x, mj, jnp.float32(0.0))
        return tuple(acc)
      return row_body

    NC_W = IMGS_PER_W * NCHUNK

    def start(k, slot):
        img = wid * IMGS_PER_W + k // NCHUNK
        b = (img * 43691) >> 19
        tt = img - b * T
        y0 = pl.multiple_of((k % NCHUNK) * YW, YW)
        pb, tb, mb = bufs[slot]
        sem = sems[slot]
        pltpu.async_copy(p_hbm.at[b, tt, pl.ds(y0, YW), :], pb, sem)
        pltpu.async_copy(t_hbm.at[b, tt, pl.ds(y0, YW), :], tb, sem)
        pltpu.async_copy(m_hbm.at[b, tt, pl.ds(y0, YW), :], mb, sem)

    def wait(slot):
        pb, tb, mb = bufs[slot]
        sem = sems[slot]
        src = (0, 0, pl.ds(0, YW), slice(None))
        pltpu.make_async_copy(p_hbm.at[src[0], src[1], src[2], :], pb,
                              sem).wait()
        pltpu.make_async_copy(t_hbm.at[src[0], src[1], src[2], :], tb,
                              sem).wait()
        pltpu.make_async_copy(m_hbm.at[src[0], src[1], src[2], :], mb,
                              sem).wait()

    def compute(slot, tot):
        pb, tb, mb = bufs[slot]
        accs = lax.fori_loop(0, YW, row_body_for(pb, tb, mb), (zf,) * 8)
        return tuple(t + a for t, a in zip(tot, accs))

    start(0, 0)

    def body2(i, tot):
        start(2 * i + 1, 1)
        wait(0)
        tot = compute(0, tot)

        @pl.when(2 * i + 2 < NC_W)
        def _():
            start(2 * i + 2, 0)

        wait(1)
        tot = compute(1, tot)
        return tot

    tots = lax.fori_loop(0, (NC_W + 1) // 2, body2, (zf,) * 8)
    t_abs, t_sq, t_cnt, t_xor = (tots[i] + tots[4 + i] for i in range(4))

    obuf[0, :] = t_cnt
    obuf[1, :] = t_abs
    obuf[2, :] = t_sq
    obuf[3, :] = t_xor
    pltpu.sync_copy(obuf, out_hbm.at[wid])


def kernel(preds, target, mask):
    mf = mask.astype(jnp.float32)

    parts = _sc_masked_sums(preds, target, mf)

    cnt = jnp.sum(parts[:, 0, :])
    sabs = jnp.sum(parts[:, 1, :])
    ssq = jnp.sum(parts[:, 2, :])
    xcnt = jnp.sum(parts[:, 3, :])

    masked_mae = sabs / cnt
    masked_mse = ssq / cnt
    masked_rmse = jnp.sqrt(masked_mse)
    iiee = xcnt * jnp.float32(625.0 / 1000000.0) / jnp.float32(96.0)
    bacc = jnp.float32(1.0) - iiee / jnp.float32(27207.0 * 625.0 / 1000000.0)
    return (masked_mae, masked_rmse, iiee, bacc, masked_mae)
